# Initial kernel scaffold; baseline (speedup 1.0000x reference)
import jax
import jax.numpy as jnp
from jax.experimental import pallas as pl


def kernel(env, experts_predictions, We0, be0, We1, be1, We2, be2, We3, be3, We4, be4, Wg1, bg1, Wg2, bg2, Wg3, bg3):
    raise NotImplementedError("write your pallas kernel here")


# speedup vs baseline: 2.5316x; 2.5316x over previous
import jax
import jax.numpy as jnp
from jax.experimental import pallas as pl

LANES = 128
BM = 512


def _moe_kernel(env_ref, ep_ref, wcat_ref, b1_ref, sel_ref, wg2_ref,
                bg2_ref, wg3_ref, bg3_ref, out_ref):
    x = env_ref[:]
    acc = jnp.dot(x, wcat_ref[:], preferred_element_type=jnp.float32)
    acc = acc + jnp.dot(ep_ref[:], sel_ref[:], preferred_element_type=jnp.float32)
    acc = acc + b1_ref[:]

    h1 = jnp.maximum(acc, 0.0)
    h2 = jnp.maximum(
        jnp.dot(h1, wg2_ref[:], preferred_element_type=jnp.float32) + bg2_ref[:], 0.0)
    logits = jnp.dot(h2, wg3_ref[:], preferred_element_type=jnp.float32) + bg3_ref[:]

    bm = logits.shape[0]
    lane = jax.lax.broadcasted_iota(jnp.int32, (bm, LANES), 1)
    in8 = lane < 8
    lm = jnp.where(in8, logits, jnp.float32(-1e30))
    mx = jnp.max(lm, axis=1, keepdims=True)
    ex = jnp.exp(lm - mx)
    g = ex / jnp.sum(ex, axis=1, keepdims=True)

    vals = []
    outs = []
    for _ in range(3):
        mk = jnp.max(g, axis=1, keepdims=True)
        idx = jnp.min(jnp.where(g == mk, lane, LANES), axis=1, keepdims=True)
        onehot = lane == idx
        vals.append(mk)
        outs.append(jnp.sum(jnp.where(onehot, acc, 0.0), axis=1, keepdims=True))
        g = jnp.where(onehot, jnp.float32(-1.0), g)

    vmx = jnp.maximum(jnp.maximum(vals[0], vals[1]), vals[2])
    e0 = jnp.exp(vals[0] - vmx)
    e1 = jnp.exp(vals[1] - vmx)
    e2 = jnp.exp(vals[2] - vmx)
    mixed = (outs[0] * e0 + outs[1] * e1 + outs[2] * e2) / (e0 + e1 + e2)
    out_ref[:] = mixed


def _run(env, ep, wcat, b1, sel, wg2p, bg2p, wg3p, bg3p, interpret=False):
    B, D = env.shape
    grid = (B // BM,)
    return pl.pallas_call(
        _moe_kernel,
        grid=grid,
        in_specs=[
            pl.BlockSpec((BM, D), lambda i: (i, 0)),
            pl.BlockSpec((BM, 8), lambda i: (i, 0)),
            pl.BlockSpec((D, LANES), lambda i: (0, 0)),
            pl.BlockSpec((1, LANES), lambda i: (0, 0)),
            pl.BlockSpec((8, LANES), lambda i: (0, 0)),
            pl.BlockSpec((LANES, LANES), lambda i: (0, 0)),
            pl.BlockSpec((1, LANES), lambda i: (0, 0)),
            pl.BlockSpec((LANES, LANES), lambda i: (0, 0)),
            pl.BlockSpec((1, LANES), lambda i: (0, 0)),
        ],
        out_specs=pl.BlockSpec((BM, 1), lambda i: (i, 0)),
        out_shape=jax.ShapeDtypeStruct((B, 1), jnp.float32),
        interpret=interpret,
    )(env, ep, wcat, b1, sel, wg2p, bg2p, wg3p, bg3p)


def kernel(env, experts_predictions, We0, be0, We1, be1, We2, be2, We3, be3,
           We4, be4, Wg1, bg1, Wg2, bg2, Wg3, bg3):
    D = env.shape[1]
    H = Wg1.shape[1]
    f32 = jnp.float32

    wcat = jnp.concatenate([
        We0, We1, We2, We3, We4[:, 0:1],
        jnp.zeros((D, 3), f32), Wg1,
        jnp.zeros((D, LANES - 8 - H), f32)], axis=1)
    b1 = jnp.concatenate([
        be0, be1, be2, be3, be4[0:1],
        jnp.zeros((3,), f32), bg1,
        jnp.zeros((LANES - 8 - H,), f32)]).reshape(1, LANES)
    sel = jnp.zeros((8, LANES), f32)
    sel = sel.at[0, 5].set(1.0).at[1, 6].set(1.0).at[2, 7].set(1.0)
    wg2p = jnp.zeros((LANES, LANES), f32).at[8:8 + H, 0:H].set(Wg2)
    bg2p = jnp.zeros((1, LANES), f32).at[0, 0:H].set(bg2)
    wg3p = jnp.zeros((LANES, LANES), f32).at[0:H, 0:8].set(Wg3)
    bg3p = jnp.zeros((1, LANES), f32).at[0, 0:8].set(bg3)
    ep = jnp.pad(experts_predictions, ((0, 0), (0, 5)))

    out = _run(env, ep, wcat, b1, sel, wg2p, bg2p, wg3p, bg3p)
    return out[:, 0]


# speedup vs baseline: 2.8008x; 1.1063x over previous
import jax
import jax.numpy as jnp
from jax.experimental import pallas as pl

LANES = 128
BM = 1024


def _moe_kernel(env_ref, ep_ref, wcat_ref, b1_ref, sel_ref, wg2_ref,
                bg2_ref, wg3_ref, bg3_ref, out_ref):
    x = env_ref[:]
    acc = jnp.dot(x, wcat_ref[:], preferred_element_type=jnp.float32)
    acc = acc + jnp.dot(ep_ref[:], sel_ref[:], preferred_element_type=jnp.float32)
    acc = acc + b1_ref[:]

    h1 = jnp.maximum(acc, 0.0)
    h2 = jnp.maximum(
        jnp.dot(h1, wg2_ref[:], preferred_element_type=jnp.float32) + bg2_ref[:], 0.0)
    logits = jnp.dot(h2, wg3_ref[:], preferred_element_type=jnp.float32) + bg3_ref[:]

    bm = logits.shape[0]
    lane = jax.lax.broadcasted_iota(jnp.int32, (bm, LANES), 1)
    in8 = lane < 8
    lm = jnp.where(in8, logits, jnp.float32(-1e30))
    mx = jnp.max(lm, axis=1, keepdims=True)
    ex = jnp.exp(lm - mx)
    g = ex / jnp.sum(ex, axis=1, keepdims=True)

    vals = []
    outs = []
    for _ in range(3):
        mk = jnp.max(g, axis=1, keepdims=True)
        idx = jnp.min(jnp.where(g == mk, lane, LANES), axis=1, keepdims=True)
        onehot = lane == idx
        vals.append(mk)
        outs.append(jnp.sum(jnp.where(onehot, acc, 0.0), axis=1, keepdims=True))
        g = jnp.where(onehot, jnp.float32(-1.0), g)

    vmx = jnp.maximum(jnp.maximum(vals[0], vals[1]), vals[2])
    e0 = jnp.exp(vals[0] - vmx)
    e1 = jnp.exp(vals[1] - vmx)
    e2 = jnp.exp(vals[2] - vmx)
    mixed = (outs[0] * e0 + outs[1] * e1 + outs[2] * e2) / (e0 + e1 + e2)
    out_ref[:] = mixed


def _run(env, ep, wcat, b1, sel, wg2p, bg2p, wg3p, bg3p, interpret=False):
    B, D = env.shape
    grid = (B // BM,)
    return pl.pallas_call(
        _moe_kernel,
        grid=grid,
        in_specs=[
            pl.BlockSpec((BM, D), lambda i: (i, 0)),
            pl.BlockSpec((BM, 8), lambda i: (i, 0)),
            pl.BlockSpec((D, LANES), lambda i: (0, 0)),
            pl.BlockSpec((1, LANES), lambda i: (0, 0)),
            pl.BlockSpec((8, LANES), lambda i: (0, 0)),
            pl.BlockSpec((LANES, LANES), lambda i: (0, 0)),
            pl.BlockSpec((1, LANES), lambda i: (0, 0)),
            pl.BlockSpec((LANES, LANES), lambda i: (0, 0)),
            pl.BlockSpec((1, LANES), lambda i: (0, 0)),
        ],
        out_specs=pl.BlockSpec((BM, 1), lambda i: (i, 0)),
        out_shape=jax.ShapeDtypeStruct((B, 1), jnp.float32),
        interpret=interpret,
    )(env, ep, wcat, b1, sel, wg2p, bg2p, wg3p, bg3p)


def kernel(env, experts_predictions, We0, be0, We1, be1, We2, be2, We3, be3,
           We4, be4, Wg1, bg1, Wg2, bg2, Wg3, bg3):
    D = env.shape[1]
    H = Wg1.shape[1]
    f32 = jnp.float32

    wcat = jnp.concatenate([
        We0, We1, We2, We3, We4[:, 0:1],
        jnp.zeros((D, 3), f32), Wg1,
        jnp.zeros((D, LANES - 8 - H), f32)], axis=1)
    b1 = jnp.concatenate([
        be0, be1, be2, be3, be4[0:1],
        jnp.zeros((3,), f32), bg1,
        jnp.zeros((LANES - 8 - H,), f32)]).reshape(1, LANES)
    sel = jnp.zeros((8, LANES), f32)
    sel = sel.at[0, 5].set(1.0).at[1, 6].set(1.0).at[2, 7].set(1.0)
    wg2p = jnp.zeros((LANES, LANES), f32).at[8:8 + H, 0:H].set(Wg2)
    bg2p = jnp.zeros((1, LANES), f32).at[0, 0:H].set(bg2)
    wg3p = jnp.zeros((LANES, LANES), f32).at[0:H, 0:8].set(Wg3)
    bg3p = jnp.zeros((1, LANES), f32).at[0, 0:8].set(bg3)
    ep = jnp.pad(experts_predictions, ((0, 0), (0, 5)))

    out = _run(env, ep, wcat, b1, sel, wg2p, bg2p, wg3p, bg3p)
    return out[:, 0]


# speedup vs baseline: 2.8733x; 1.0259x over previous
import jax
import jax.numpy as jnp
from jax.experimental import pallas as pl

LANES = 128
BM = 2048


def _moe_kernel(env_ref, ep_ref, wcat_ref, b1_ref, sel_ref, wg2_ref,
                bg2_ref, wg3_ref, bg3_ref, out_ref):
    x = env_ref[:]
    acc = jnp.dot(x, wcat_ref[:], preferred_element_type=jnp.float32)
    acc = acc + jnp.dot(ep_ref[:], sel_ref[:], preferred_element_type=jnp.float32)
    acc = acc + b1_ref[:]

    h1 = jnp.maximum(acc, 0.0)
    h2 = jnp.maximum(
        jnp.dot(h1, wg2_ref[:], preferred_element_type=jnp.float32) + bg2_ref[:], 0.0)
    logits = jnp.dot(h2, wg3_ref[:], preferred_element_type=jnp.float32) + bg3_ref[:]

    bm = logits.shape[0]
    lane = jax.lax.broadcasted_iota(jnp.int32, (bm, LANES), 1)
    in8 = lane < 8
    lm = jnp.where(in8, logits, jnp.float32(-1e30))
    mx = jnp.max(lm, axis=1, keepdims=True)
    ex = jnp.exp(lm - mx)
    g = ex / jnp.sum(ex, axis=1, keepdims=True)

    vals = []
    outs = []
    for _ in range(3):
        mk = jnp.max(g, axis=1, keepdims=True)
        idx = jnp.min(jnp.where(g == mk, lane, LANES), axis=1, keepdims=True)
        onehot = lane == idx
        vals.append(mk)
        outs.append(jnp.sum(jnp.where(onehot, acc, 0.0), axis=1, keepdims=True))
        g = jnp.where(onehot, jnp.float32(-1.0), g)

    vmx = jnp.maximum(jnp.maximum(vals[0], vals[1]), vals[2])
    e0 = jnp.exp(vals[0] - vmx)
    e1 = jnp.exp(vals[1] - vmx)
    e2 = jnp.exp(vals[2] - vmx)
    mixed = (outs[0] * e0 + outs[1] * e1 + outs[2] * e2) / (e0 + e1 + e2)
    out_ref[:] = mixed


def _run(env, ep, wcat, b1, sel, wg2p, bg2p, wg3p, bg3p, interpret=False):
    B, D = env.shape
    grid = (B // BM,)
    return pl.pallas_call(
        _moe_kernel,
        grid=grid,
        in_specs=[
            pl.BlockSpec((BM, D), lambda i: (i, 0)),
            pl.BlockSpec((BM, 8), lambda i: (i, 0)),
            pl.BlockSpec((D, LANES), lambda i: (0, 0)),
            pl.BlockSpec((1, LANES), lambda i: (0, 0)),
            pl.BlockSpec((8, LANES), lambda i: (0, 0)),
            pl.BlockSpec((LANES, LANES), lambda i: (0, 0)),
            pl.BlockSpec((1, LANES), lambda i: (0, 0)),
            pl.BlockSpec((LANES, LANES), lambda i: (0, 0)),
            pl.BlockSpec((1, LANES), lambda i: (0, 0)),
        ],
        out_specs=pl.BlockSpec((BM, 1), lambda i: (i, 0)),
        out_shape=jax.ShapeDtypeStruct((B, 1), jnp.float32),
        interpret=interpret,
    )(env, ep, wcat, b1, sel, wg2p, bg2p, wg3p, bg3p)


def kernel(env, experts_predictions, We0, be0, We1, be1, We2, be2, We3, be3,
           We4, be4, Wg1, bg1, Wg2, bg2, Wg3, bg3):
    D = env.shape[1]
    H = Wg1.shape[1]
    f32 = jnp.float32

    wcat = jnp.concatenate([
        We0, We1, We2, We3, We4[:, 0:1],
        jnp.zeros((D, 3), f32), Wg1,
        jnp.zeros((D, LANES - 8 - H), f32)], axis=1)
    b1 = jnp.concatenate([
        be0, be1, be2, be3, be4[0:1],
        jnp.zeros((3,), f32), bg1,
        jnp.zeros((LANES - 8 - H,), f32)]).reshape(1, LANES)
    sel = jnp.zeros((8, LANES), f32)
    sel = sel.at[0, 5].set(1.0).at[1, 6].set(1.0).at[2, 7].set(1.0)
    wg2p = jnp.zeros((LANES, LANES), f32).at[8:8 + H, 0:H].set(Wg2)
    bg2p = jnp.zeros((1, LANES), f32).at[0, 0:H].set(bg2)
    wg3p = jnp.zeros((LANES, LANES), f32).at[0:H, 0:8].set(Wg3)
    bg3p = jnp.zeros((1, LANES), f32).at[0, 0:8].set(bg3)
    ep = jnp.pad(experts_predictions, ((0, 0), (0, 5)))

    out = _run(env, ep, wcat, b1, sel, wg2p, bg2p, wg3p, bg3p)
    return out[:, 0]
